# baseline (device time: 47191 ns/iter reference)
import jax
import jax.numpy as jnp
from jax import lax
from jax.experimental import pallas as pl
from jax.experimental.pallas import tpu as pltpu

N_DEV = 8
HQ_PER = 8
DH = 128
SQ = 1024
D_MODEL = 1024
BLK = 64
N_CLASS = 4
BLOCKS_PER_CLASS = 4
SCALE = 0.08838834764831843
BF16 = jnp.bfloat16
F32 = jnp.float32

WID = [384, 384, 256]
COL0 = [0, 384, 768]
SIZES = [512, 256, 128]
REG = [0, 512, 768]


def kernel(x, Wq, K_ext, V_ext, Wo):
    def body(x_ref, wq_ref, k_hbm, v_hbm, wo_ref, out_ref,
             k_ref, v_ref, q_ref, ctx_ref, wo_bf_ref,
             obf0, obf1, obf2, rs0, rs1, rs2, ag0, ag1, ag2,
             copy_sems, send_sems, recv_sems):
        me = lax.axis_index("i")
        h0 = me * HQ_PER
        obf_refs = [obf0, obf1, obf2]
        rs_refs = [rs0, rs1, rs2]
        ag_refs = [ag0, ag1, ag2]

        t = lax.rem(me, 4)
        b_x = jnp.where((t == 1) | (t == 2), 1, 0)
        b_y = jnp.where(t >= 2, 1, 0)
        b_z = lax.div(me, 4)
        p_x = me + 1 - 2 * lax.rem(me, 2)
        p_y = (me - t) + (3 - t)
        p_z = me + 4 - 8 * b_z
        dx, dy, dz = (p_x, b_x), (p_y, b_y), (p_z, b_z)
        dims_s = [[dx, dy, dz], [dy, dz, dx], [dz, dx, dy]]

        kv_copies = []
        for h in range(HQ_PER):
            kc = pltpu.make_async_copy(
                k_hbm.at[0, :, h0 + h, :], k_ref.at[h], copy_sems.at[h])
            vc = pltpu.make_async_copy(
                v_hbm.at[0, :, h0 + h, :], v_ref.at[h],
                copy_sems.at[HQ_PER + h])
            kc.start()
            vc.start()
            kv_copies.append((kc, vc))

        barrier = pltpu.get_barrier_semaphore()
        for nbr in (p_x, p_y, p_z):
            pl.semaphore_signal(barrier, inc=1, device_id=(nbr,),
                                device_id_type=pl.DeviceIdType.MESH)
        pl.semaphore_wait(barrier, 3)

        wo_bf_ref[:, :] = wo_ref[:, :].astype(BF16)

        q_ref[:, :] = (jnp.dot(
            x_ref[0].astype(BF16), wq_ref[:, :].astype(BF16),
            preferred_element_type=F32) * SCALE).astype(BF16)

        for h in range(HQ_PER):
            for cp in kv_copies[h]:
                cp.wait()
            for c in range(N_CLASS):
                rows = [(c + N_CLASS * b) * BLK
                        for b in range(BLOCKS_PER_CLASS)]
                qc = jnp.concatenate(
                    [q_ref[r:r + BLK, h * DH:(h + 1) * DH] for r in rows], 0)
                kc = jnp.concatenate(
                    [k_ref[h, r:r + BLK, :] for r in rows], 0).astype(BF16)
                vc = jnp.concatenate(
                    [v_ref[h, r:r + BLK, :] for r in rows], 0).astype(BF16)
                s = lax.dot_general(
                    qc, kc, (((1,), (1,)), ((), ())),
                    preferred_element_type=F32)
                e = jnp.exp(s)
                w = (e / jnp.sum(e, axis=1, keepdims=True)).astype(BF16)
                ctx = jnp.dot(w, vc, preferred_element_type=F32)
                for b, r in enumerate(rows):
                    ctx_ref[r:r + BLK, h * DH:(h + 1) * DH] = \
                        ctx[b * BLK:(b + 1) * BLK, :].astype(BF16)

        o_ref = out_ref.at[0]

        def ocols(sch, r0, sz):
            return o_ref.at[pl.ds(r0, sz), COL0[sch]:COL0[sch] + WID[sch]]

        def gemm_rows(r0, sch, nrows):
            ocols(sch, r0, nrows)[:, :] = jnp.dot(
                ctx_ref[pl.ds(r0, nrows), :],
                wo_bf_ref[:, COL0[sch]:COL0[sch] + WID[sch]],
                preferred_element_type=F32)

        def xchg(src, dst, nbr, ssem, rsem):
            return pltpu.make_async_remote_copy(
                src_ref=src, dst_ref=dst, send_sem=ssem, recv_sem=rsem,
                device_id=(nbr,), device_id_type=pl.DeviceIdType.MESH)

        kp = [[], [], []]
        snd = [[], [], []]
        ag_src = [[], [], []]
        prt = [[], [], []]
        for s in range(3):
            off = 0
            for r in range(3):
                sz = SIZES[r]
                b = dims_s[s][r][1]
                snd[s].append(off + (1 - b) * sz)
                kp[s].append(off + b * sz)
                off = kp[s][r]
            for j, r in enumerate([2, 1, 0]):
                sz = SIZES[r]
                b = dims_s[s][r][1]
                ag_src[s].append(off)
                base = off - b * sz
                prt[s].append(base + (1 - b) * sz)
                off = base

        def rs_hop(s, r):
            return xchg(obf_refs[s].at[pl.ds(snd[s][r], SIZES[r]), :],
                        rs_refs[s].at[pl.ds(REG[r], SIZES[r]), :],
                        dims_s[s][r][0],
                        send_sems.at[s, r], recv_sems.at[s, r])

        def ag_hop(s, j):
            r = 2 - j
            return xchg(obf_refs[s].at[pl.ds(ag_src[s][j], SIZES[r]), :],
                        ag_refs[s].at[pl.ds(REG[r], SIZES[r]), :],
                        dims_s[s][r][0],
                        send_sems.at[s, 3 + j], recv_sems.at[s, 3 + j])

        hops = [None, None, None]
        for s in range(3):
            gemm_rows(snd[s][0], s, 512)
            obf_refs[s][pl.ds(snd[s][0], 512), :] = \
                ocols(s, snd[s][0], 512)[:, :].astype(BF16)
            hops[s] = rs_hop(s, 0)
            hops[s].start()
        for s in range(3):
            gemm_rows(kp[s][0], s, 512)
        for r in range(3):
            sz = SIZES[r]
            reg = REG[r]
            for s in range(3):
                hops[s].wait()
                val = (ocols(s, kp[s][r], sz)[:, :]
                       + rs_refs[s][reg:reg + sz, :].astype(F32))
                ocols(s, kp[s][r], sz)[:, :] = val
                obf_refs[s][pl.ds(kp[s][r], sz), :] = val.astype(BF16)
                hops[s] = rs_hop(s, r + 1) if r < 2 else ag_hop(s, 0)
                hops[s].start()

        for j, r in enumerate([2, 1, 0]):
            sz = SIZES[r]
            reg = REG[r]
            for s in range(3):
                hops[s].wait()
                obf_refs[s][pl.ds(prt[s][j], sz), :] = \
                    ag_refs[s][reg:reg + sz, :]
                if j < 2:
                    hops[s] = ag_hop(s, j + 1)
                    hops[s].start()
                ocols(s, prt[s][j], sz)[:, :] = \
                    ag_refs[s][reg:reg + sz, :].astype(F32)

    return pl.pallas_call(
        body,
        out_shape=jax.ShapeDtypeStruct((1, SQ, D_MODEL), jnp.float32),
        in_specs=[
            pl.BlockSpec(memory_space=pltpu.VMEM),
            pl.BlockSpec(memory_space=pltpu.VMEM),
            pl.BlockSpec(memory_space=pl.ANY),
            pl.BlockSpec(memory_space=pl.ANY),
            pl.BlockSpec(memory_space=pltpu.VMEM),
        ],
        out_specs=pl.BlockSpec(memory_space=pltpu.VMEM),
        scratch_shapes=[
            pltpu.VMEM((HQ_PER, SQ, DH), F32),
            pltpu.VMEM((HQ_PER, SQ, DH), F32),
            pltpu.VMEM((SQ, HQ_PER * DH), BF16),
            pltpu.VMEM((SQ, HQ_PER * DH), BF16),
            pltpu.VMEM((D_MODEL, D_MODEL), BF16),
            pltpu.VMEM((SQ, WID[0]), BF16),
            pltpu.VMEM((SQ, WID[1]), BF16),
            pltpu.VMEM((SQ, WID[2]), BF16),
            pltpu.VMEM((896, WID[0]), BF16),
            pltpu.VMEM((896, WID[1]), BF16),
            pltpu.VMEM((896, WID[2]), BF16),
            pltpu.VMEM((896, WID[0]), BF16),
            pltpu.VMEM((896, WID[1]), BF16),
            pltpu.VMEM((896, WID[2]), BF16),
            pltpu.SemaphoreType.DMA((2 * HQ_PER,)),
            pltpu.SemaphoreType.DMA((3, 6)),
            pltpu.SemaphoreType.DMA((3, 6)),
        ],
        compiler_params=pltpu.CompilerParams(collective_id=0),
    )(x, Wq, K_ext, V_ext, Wo)


# device time: 46217 ns/iter; 1.0211x vs baseline; 1.0211x over previous
import jax
import jax.numpy as jnp
from jax import lax
from jax.experimental import pallas as pl
from jax.experimental.pallas import tpu as pltpu

N_DEV = 8
HQ_PER = 8
DH = 128
SQ = 1024
D_MODEL = 1024
BLK = 64
N_CLASS = 4
BLOCKS_PER_CLASS = 4
SCALE = 0.08838834764831843
BF16 = jnp.bfloat16
F32 = jnp.float32

WID = [384, 384, 256]
COL0 = [0, 384, 768]
SIZES = [512, 256, 128]
REG = [0, 512, 768]


def kernel(x, Wq, K_ext, V_ext, Wo):
    def body(x_ref, wq_ref, k_hbm, v_hbm, wo_ref, out_ref,
             k_ref, v_ref, q_ref, ctx_ref, wo_bf_ref,
             obf0, obf1, obf2, rs0, rs1, rs2, ag0, ag1, ag2,
             copy_sems, send_sems, recv_sems):
        me = lax.axis_index("i")
        h0 = me * HQ_PER
        obf_refs = [obf0, obf1, obf2]
        rs_refs = [rs0, rs1, rs2]
        ag_refs = [ag0, ag1, ag2]

        t = lax.rem(me, 4)
        b_x = jnp.where((t == 1) | (t == 2), 1, 0)
        b_y = jnp.where(t >= 2, 1, 0)
        b_z = lax.div(me, 4)
        p_x = me + 1 - 2 * lax.rem(me, 2)
        p_y = (me - t) + (3 - t)
        p_z = me + 4 - 8 * b_z
        dx, dy, dz = (p_x, b_x), (p_y, b_y), (p_z, b_z)
        dims_s = [[dx, dy, dz], [dy, dz, dx], [dz, dx, dy]]

        copies = []
        for h in range(HQ_PER):
            kc = pltpu.make_async_copy(
                k_hbm.at[0, :, h0 + h, :], k_ref.at[h], copy_sems.at[h])
            vc = pltpu.make_async_copy(
                v_hbm.at[0, :, h0 + h, :], v_ref.at[h],
                copy_sems.at[HQ_PER + h])
            kc.start()
            vc.start()
            copies += [kc, vc]

        barrier = pltpu.get_barrier_semaphore()
        for nbr in (p_x, p_y, p_z):
            pl.semaphore_signal(barrier, inc=1, device_id=(nbr,),
                                device_id_type=pl.DeviceIdType.MESH)
        pl.semaphore_wait(barrier, 3)

        wo_bf_ref[:, :] = wo_ref[:, :].astype(BF16)

        q_ref[:, :] = jnp.dot(
            x_ref[0].astype(BF16), wq_ref[:, :].astype(BF16),
            preferred_element_type=F32).astype(BF16)

        for c in copies:
            c.wait()

        for c in range(N_CLASS):
            rows = [(c + N_CLASS * b) * BLK for b in range(BLOCKS_PER_CLASS)]
            for h in range(HQ_PER):
                qc = jnp.concatenate(
                    [q_ref[r:r + BLK, h * DH:(h + 1) * DH] for r in rows], 0)
                kc = jnp.concatenate(
                    [k_ref[h, r:r + BLK, :] for r in rows], 0).astype(BF16)
                vc = jnp.concatenate(
                    [v_ref[h, r:r + BLK, :] for r in rows], 0).astype(BF16)
                s = lax.dot_general(
                    qc, kc, (((1,), (1,)), ((), ())),
                    preferred_element_type=F32) * SCALE
                m = jnp.max(s, axis=1, keepdims=True)
                e = jnp.exp(s - m)
                w = (e / jnp.sum(e, axis=1, keepdims=True)).astype(BF16)
                ctx = jnp.dot(w, vc, preferred_element_type=F32)
                for b, r in enumerate(rows):
                    ctx_ref[r:r + BLK, h * DH:(h + 1) * DH] = \
                        ctx[b * BLK:(b + 1) * BLK, :].astype(BF16)

        o_ref = out_ref.at[0]

        def gemm_rows(r0, sch, nrows):
            obf_refs[sch][pl.ds(r0, nrows), :] = jnp.dot(
                ctx_ref[pl.ds(r0, nrows), :],
                wo_bf_ref[:, COL0[sch]:COL0[sch] + WID[sch]],
                preferred_element_type=F32).astype(BF16)

        def xchg(src, dst, nbr, ssem, rsem):
            return pltpu.make_async_remote_copy(
                src_ref=src, dst_ref=dst, send_sem=ssem, recv_sem=rsem,
                device_id=(nbr,), device_id_type=pl.DeviceIdType.MESH)

        kp = [[], [], []]
        snd = [[], [], []]
        ag_src = [[], [], []]
        prt = [[], [], []]
        for s in range(3):
            off = 0
            for r in range(3):
                sz = SIZES[r]
                b = dims_s[s][r][1]
                snd[s].append(off + (1 - b) * sz)
                kp[s].append(off + b * sz)
                off = kp[s][r]
            for j, r in enumerate([2, 1, 0]):
                sz = SIZES[r]
                b = dims_s[s][r][1]
                ag_src[s].append(off)
                base = off - b * sz
                prt[s].append(base + (1 - b) * sz)
                off = base

        def rs_hop(s, r):
            return xchg(obf_refs[s].at[pl.ds(snd[s][r], SIZES[r]), :],
                        rs_refs[s].at[pl.ds(REG[r], SIZES[r]), :],
                        dims_s[s][r][0],
                        send_sems.at[s, r], recv_sems.at[s, r])

        def ag_hop(s, j):
            r = 2 - j
            return xchg(obf_refs[s].at[pl.ds(ag_src[s][j], SIZES[r]), :],
                        ag_refs[s].at[pl.ds(REG[r], SIZES[r]), :],
                        dims_s[s][r][0],
                        send_sems.at[s, 3 + j], recv_sems.at[s, 3 + j])

        hops = [None, None, None]
        for s in range(3):
            gemm_rows(snd[s][0], s, 512)
            hops[s] = rs_hop(s, 0)
            hops[s].start()
        for s in range(3):
            gemm_rows(kp[s][0], s, 512)
        for r in range(3):
            sz = SIZES[r]
            reg = REG[r]
            for s in range(3):
                hops[s].wait()
                obf_refs[s][pl.ds(kp[s][r], sz), :] = (
                    obf_refs[s][pl.ds(kp[s][r], sz), :]
                    + rs_refs[s][reg:reg + sz, :])
                hops[s] = rs_hop(s, r + 1) if r < 2 else ag_hop(s, 0)
                hops[s].start()

        for j, r in enumerate([2, 1, 0]):
            sz = SIZES[r]
            reg = REG[r]
            for s in range(3):
                hops[s].wait()
                obf_refs[s][pl.ds(prt[s][j], sz), :] = \
                    ag_refs[s][reg:reg + sz, :]
                if j < 2:
                    hops[s] = ag_hop(s, j + 1)
                    hops[s].start()

        for s in range(3):
            o_ref[:, COL0[s]:COL0[s] + WID[s]] = \
                obf_refs[s][:, :].astype(F32)

    return pl.pallas_call(
        body,
        out_shape=jax.ShapeDtypeStruct((1, SQ, D_MODEL), jnp.float32),
        in_specs=[
            pl.BlockSpec(memory_space=pltpu.VMEM),
            pl.BlockSpec(memory_space=pltpu.VMEM),
            pl.BlockSpec(memory_space=pl.ANY),
            pl.BlockSpec(memory_space=pl.ANY),
            pl.BlockSpec(memory_space=pltpu.VMEM),
        ],
        out_specs=pl.BlockSpec(memory_space=pltpu.VMEM),
        scratch_shapes=[
            pltpu.VMEM((HQ_PER, SQ, DH), F32),
            pltpu.VMEM((HQ_PER, SQ, DH), F32),
            pltpu.VMEM((SQ, HQ_PER * DH), BF16),
            pltpu.VMEM((SQ, HQ_PER * DH), BF16),
            pltpu.VMEM((D_MODEL, D_MODEL), BF16),
            pltpu.VMEM((SQ, WID[0]), BF16),
            pltpu.VMEM((SQ, WID[1]), BF16),
            pltpu.VMEM((SQ, WID[2]), BF16),
            pltpu.VMEM((896, WID[0]), BF16),
            pltpu.VMEM((896, WID[1]), BF16),
            pltpu.VMEM((896, WID[2]), BF16),
            pltpu.VMEM((896, WID[0]), BF16),
            pltpu.VMEM((896, WID[1]), BF16),
            pltpu.VMEM((896, WID[2]), BF16),
            pltpu.SemaphoreType.DMA((2 * HQ_PER,)),
            pltpu.SemaphoreType.DMA((3, 6)),
            pltpu.SemaphoreType.DMA((3, 6)),
        ],
        compiler_params=pltpu.CompilerParams(collective_id=0),
    )(x, Wq, K_ext, V_ext, Wo)
